# baseline (speedup 1.0000x reference)
import functools

import jax
import jax.numpy as jnp
from jax import lax
from jax.experimental import pallas as pl
from jax.experimental.pallas import tpu as pltpu
from jax.experimental.pallas import tpu_sc as plsc

VOCAB = 1000
EMBED_DIM = 128
BATCH = 1024
SEQ = 50

TOK = BATCH * SEQ
NC, NS = 2, 16
NW = NC * NS
BPW = TOK // NW
CH = 40
NCHUNK = BPW // CH


def _pmat_body(table_ref, w_ref, b_ref, p_ref):
    p_ref[...] = lax.dot_general(
        table_ref[...], w_ref[...],
        dimension_numbers=(((1,), (1,)), ((), ())),
        preferred_element_type=jnp.float32,
    ) + b_ref[...]


def _compute_p(table, w, b):
    return pl.pallas_call(
        _pmat_body,
        out_shape=jax.ShapeDtypeStruct((VOCAB, VOCAB), jnp.float32),
    )(table, w, b.reshape(1, VOCAB))


_SC_MESH = plsc.VectorSubcoreMesh(
    core_axis_name="c", subcore_axis_name="s", num_cores=NC, num_subcores=NS)


@functools.partial(
    pl.kernel,
    out_type=jax.ShapeDtypeStruct((TOK, VOCAB), jnp.float32),
    mesh=_SC_MESH,
    scratch_types=[
        pltpu.VMEM((2, CH), jnp.int32),
        pltpu.VMEM((2, CH, VOCAB), jnp.float32),
        pltpu.SemaphoreType.DMA,
        pltpu.SemaphoreType.DMA,
        pltpu.SemaphoreType.DMA,
        pltpu.SemaphoreType.DMA,
    ],
    compiler_params=pltpu.CompilerParams(use_tc_tiling_on_sc=False),
)
def _sc_gather(p_hbm, idx_hbm, out_hbm, idx_v, rows_v, g0, g1, w0, w1):
    wid = lax.axis_index("s") * NC + lax.axis_index("c")
    base = wid * BPW

    gsem = (g0, g1)
    wsem = (w0, w1)

    def load_idx(c, slot):
        pltpu.sync_copy(idx_hbm.at[pl.ds(base + c * CH, CH)], idx_v.at[slot])

    def gather(slot):
        pltpu.async_copy(p_hbm.at[idx_v.at[slot]], rows_v.at[slot], gsem[slot])

    def wait_gather(slot):
        pltpu.make_async_copy(
            p_hbm.at[idx_v.at[slot]], rows_v.at[slot], gsem[slot]).wait()

    def write(c, slot):
        pltpu.async_copy(
            rows_v.at[slot], out_hbm.at[pl.ds(base + c * CH, CH)], wsem[slot])

    def wait_write(slot):
        pltpu.make_async_copy(
            rows_v.at[slot], out_hbm.at[pl.ds(base, CH)], wsem[slot]).wait()

    load_idx(0, 0)
    gather(0)
    load_idx(1, 1)
    gather(1)
    wait_gather(0)
    write(0, 0)

    def step(p, carry):
        a = 2 * p
        wait_write(0)
        load_idx(a, 0)
        gather(0)
        wait_gather(1)
        write(a - 1, 1)
        wait_write(1)
        load_idx(a + 1, 1)
        gather(1)
        wait_gather(0)
        write(a, 0)
        return carry

    lax.fori_loop(1, NCHUNK // 2, step, 0)

    wait_gather(1)
    write(NCHUNK - 1, 1)
    wait_write(0)
    wait_write(1)


@jax.jit
def kernel(x, table, W, b):
    p = _compute_p(table, W, b)
    out = _sc_gather(p, x.reshape(TOK))
    return out.reshape(BATCH, SEQ, VOCAB)


# speedup vs baseline: 1.6842x; 1.6842x over previous
import functools

import jax
import jax.numpy as jnp
from jax import lax
from jax.experimental import pallas as pl
from jax.experimental.pallas import tpu as pltpu
from jax.experimental.pallas import tpu_sc as plsc

VOCAB = 1000
EMBED_DIM = 128
BATCH = 1024
SEQ = 50

NC, NS = 2, 16
NW = NC * NS
RPW = BATCH // NW
NJ = 8
TAIL = VOCAB - 7 * 128


def _pmat_body(table_ref, w_ref, b_ref, *p_refs):
    p = lax.dot_general(
        table_ref[...], w_ref[...],
        dimension_numbers=(((1,), (1,)), ((), ())),
        preferred_element_type=jnp.float32,
    ) + b_ref[...]
    for j in range(NJ):
        p_refs[j][...] = p[:, j * 128:(j + 1) * 128]


def _compute_p(table, w, b):
    w_pad = jnp.pad(w, ((0, NJ * 128 - VOCAB), (0, 0)))
    b_pad = jnp.pad(b, (0, NJ * 128 - VOCAB)).reshape(1, NJ * 128)
    return pl.pallas_call(
        _pmat_body,
        out_shape=[jax.ShapeDtypeStruct((VOCAB, 128), jnp.float32)
                   for _ in range(NJ)],
    )(table, w_pad, b_pad)


_SC_MESH = plsc.VectorSubcoreMesh(
    core_axis_name="c", subcore_axis_name="s", num_cores=NC, num_subcores=NS)


@functools.partial(
    pl.kernel,
    out_type=jax.ShapeDtypeStruct((BATCH, SEQ, VOCAB), jnp.float32),
    mesh=_SC_MESH,
    scratch_types=[
        pltpu.VMEM((RPW, SEQ), jnp.int32),
        pltpu.VMEM((2, NJ, SEQ, 128), jnp.float32),
        pltpu.VMEM((SEQ, TAIL), jnp.float32),
        pltpu.SemaphoreType.DMA,
        pltpu.SemaphoreType.DMA,
        pltpu.SemaphoreType.DMA,
        pltpu.SemaphoreType.DMA,
        pltpu.SemaphoreType.DMA,
    ],
    compiler_params=pltpu.CompilerParams(use_tc_tiling_on_sc=True, needs_layout_passes=False),
)
def _sc_gather(p0, p1, p2, p3, p4, p5, p6, p7, idx_hbm, out_hbm,
               idx_v, rows_v, tail_v, g0, g1, w0, w1, wt):
    ps = (p0, p1, p2, p3, p4, p5, p6, p7)
    wid = lax.axis_index("s") * NC + lax.axis_index("c")
    row0 = wid * RPW

    gsem = (g0, g1)
    wsem = (w0, w1)

    pltpu.sync_copy(idx_hbm.at[pl.ds(row0, RPW)], idx_v)

    def gather(i, slot):
        idx = idx_v.at[i]
        for j in range(NJ):
            pltpu.async_copy(ps[j].at[idx], rows_v.at[slot, j], gsem[slot])

    def wait_gather(slot):
        for j in range(NJ):
            pltpu.make_async_copy(
                ps[j].at[idx_v.at[0]], rows_v.at[slot, j],
                gsem[slot]).wait()

    lanes = (NJ - 2) * 16 + lax.iota(jnp.int32, 16)

    def fill_tail(slot):
        src_t = rows_v.at[slot, NJ - 1]
        for r in range(SEQ):
            for c in range(TAIL // 16):
                tail_v[r, c * 16:(c + 1) * 16] = src_t[r, c * 16:(c + 1) * 16]
            rsplat = jnp.full((16,), r, jnp.int32)
            v = plsc.load_gather(src_t, [rsplat, lanes])
            plsc.store_scatter(tail_v, [rsplat, lanes], v)

    def write(i, slot):
        b = row0 + i
        for j in range(NJ - 1):
            pltpu.async_copy(
                rows_v.at[slot, j], out_hbm.at[b, :, pl.ds(j * 128, 128)],
                wsem[slot])
        pltpu.async_copy(
            tail_v, out_hbm.at[b, :, pl.ds((NJ - 1) * 128, TAIL)], wt)

    def wait_write(slot):
        for j in range(NJ - 1):
            pltpu.make_async_copy(
                rows_v.at[slot, j], out_hbm.at[row0, :, pl.ds(j * 128, 128)],
                wsem[slot]).wait()

    def wait_tail():
        pltpu.make_async_copy(
            tail_v, out_hbm.at[row0, :, pl.ds((NJ - 1) * 128, TAIL)],
            wt).wait()

    gather(0, 0)
    gather(1, 1)
    wait_gather(0)
    fill_tail(0)
    write(0, 0)

    def step(p, carry):
        i = 2 * p
        wait_write(0)
        gather(i, 0)
        wait_gather(1)
        wait_tail()
        fill_tail(1)
        write(i - 1, 1)
        wait_write(1)
        gather(i + 1, 1)
        wait_gather(0)
        wait_tail()
        fill_tail(0)
        write(i, 0)
        return carry

    lax.fori_loop(1, RPW // 2, step, 0)

    wait_gather(1)
    wait_tail()
    fill_tail(1)
    write(RPW - 1, 1)
    wait_write(0)
    wait_write(1)
    wait_tail()


@jax.jit
def kernel(x, table, W, b):
    ps = _compute_p(table, W, b)
    return _sc_gather(*ps, x)


# speedup vs baseline: 4.7345x; 2.8112x over previous
import functools

import jax
import jax.numpy as jnp
from jax import lax
from jax.experimental import pallas as pl
from jax.experimental.pallas import tpu as pltpu
from jax.experimental.pallas import tpu_sc as plsc

VOCAB = 1000
EMBED_DIM = 128
BATCH = 1024
SEQ = 50

NC, NS = 2, 16
NW = NC * NS
RPW = BATCH // NW

_SC_MESH = plsc.VectorSubcoreMesh(
    core_axis_name="c", subcore_axis_name="s", num_cores=NC, num_subcores=NS)


@functools.partial(
    pl.kernel,
    out_type=jax.ShapeDtypeStruct((SEQ, BATCH, EMBED_DIM), jnp.float32),
    mesh=_SC_MESH,
    scratch_types=[
        pltpu.VMEM((RPW, SEQ), jnp.int32),
        pltpu.VMEM((SEQ, RPW), jnp.int32),
        pltpu.VMEM((2, RPW, EMBED_DIM), jnp.float32),
        pltpu.SemaphoreType.DMA,
        pltpu.SemaphoreType.DMA,
        pltpu.SemaphoreType.DMA,
        pltpu.SemaphoreType.DMA,
    ],
    compiler_params=pltpu.CompilerParams(
        use_tc_tiling_on_sc=True, needs_layout_passes=False),
)
def _sc_embed(table_hbm, idx_hbm, g_hbm, idx_v, idxT_v, rows_v, g0, g1, w0, w1):
    wid = lax.axis_index("s") * NC + lax.axis_index("c")
    b0 = wid * RPW

    gsem = (g0, g1)
    wsem = (w0, w1)

    pltpu.sync_copy(idx_hbm.at[pl.ds(b0, RPW)], idx_v)
    for h in range(RPW // 16):
        rows16 = 16 * h + lax.iota(jnp.int32, 16)
        for s in range(SEQ):
            v = plsc.load_gather(idx_v, [rows16, jnp.full((16,), s, jnp.int32)])
            idxT_v[s, 16 * h:16 * h + 16] = v

    def gather(s, slot):
        pltpu.async_copy(table_hbm.at[idxT_v.at[s]], rows_v.at[slot],
                         gsem[slot])

    def wait_gather(slot):
        pltpu.make_async_copy(table_hbm.at[idxT_v.at[0]], rows_v.at[slot],
                              gsem[slot]).wait()

    def write(s, slot):
        pltpu.async_copy(rows_v.at[slot], g_hbm.at[s, pl.ds(b0, RPW)],
                         wsem[slot])

    def wait_write(slot):
        pltpu.make_async_copy(rows_v.at[slot], g_hbm.at[0, pl.ds(b0, RPW)],
                              wsem[slot]).wait()

    gather(0, 0)
    gather(1, 1)
    wait_gather(0)
    write(0, 0)

    def step(p, carry):
        s = 2 * p
        wait_write(0)
        gather(s, 0)
        wait_gather(1)
        write(s - 1, 1)
        wait_write(1)
        gather(s + 1, 1)
        wait_gather(0)
        write(s, 0)
        return carry

    lax.fori_loop(1, SEQ // 2, step, 0)

    wait_gather(1)
    write(SEQ - 1, 1)
    wait_write(0)
    wait_write(1)


def _proj_body(w_ref, b_ref, g_ref, y_ref):
    g = g_ref[0].astype(jnp.bfloat16)
    acc = lax.dot_general(
        w_ref[...], g,
        dimension_numbers=(((1,), (1,)), ((), ())),
        preferred_element_type=jnp.float32,
    )
    y_ref[0] = acc + b_ref[...]


def _project(w, b, g):
    return pl.pallas_call(
        _proj_body,
        grid=(SEQ,),
        in_specs=[
            pl.BlockSpec((VOCAB, EMBED_DIM), lambda s: (0, 0)),
            pl.BlockSpec((VOCAB, 1), lambda s: (0, 0)),
            pl.BlockSpec((1, BATCH, EMBED_DIM), lambda s: (s, 0, 0)),
        ],
        out_specs=pl.BlockSpec((1, VOCAB, BATCH), lambda s: (s, 0, 0)),
        out_shape=jax.ShapeDtypeStruct((SEQ, VOCAB, BATCH), jnp.float32),
    )(w, b.reshape(VOCAB, 1), g)


@jax.jit
def kernel(x, table, W, b):
    g = _sc_embed(table, x)
    y = _project(W.astype(jnp.bfloat16), b, g)
    return jnp.transpose(y, (2, 0, 1))


# speedup vs baseline: 4.9557x; 1.0467x over previous
import functools

import jax
import jax.numpy as jnp
from jax import lax
from jax.experimental import pallas as pl
from jax.experimental.pallas import tpu as pltpu
from jax.experimental.pallas import tpu_sc as plsc

VOCAB = 1000
EMBED_DIM = 128
BATCH = 1024
SEQ = 50

NC, NS = 2, 16
NW = NC * NS
RPW = BATCH // NW

CHUNKS = ((0, 26), (26, 24))

_SC_MESH = plsc.VectorSubcoreMesh(
    core_axis_name="c", subcore_axis_name="s", num_cores=NC, num_subcores=NS)


def _make_sc_embed(s0, sch):
    @functools.partial(
        pl.kernel,
        out_type=jax.ShapeDtypeStruct((sch, BATCH, EMBED_DIM), jnp.float32),
        mesh=_SC_MESH,
        scratch_types=[
            pltpu.VMEM((RPW, SEQ), jnp.int32),
            pltpu.VMEM((sch, RPW), jnp.int32),
            pltpu.VMEM((2, RPW, EMBED_DIM), jnp.float32),
            pltpu.SemaphoreType.DMA,
            pltpu.SemaphoreType.DMA,
            pltpu.SemaphoreType.DMA,
            pltpu.SemaphoreType.DMA,
        ],
        compiler_params=pltpu.CompilerParams(
            use_tc_tiling_on_sc=True, needs_layout_passes=False),
    )
    def _sc_embed(table_hbm, idx_hbm, g_hbm,
                  idx_v, idxT_v, rows_v, g0, g1, w0, w1):
        wid = lax.axis_index("s") * NC + lax.axis_index("c")
        b0 = wid * RPW

        gsem = (g0, g1)
        wsem = (w0, w1)

        pltpu.sync_copy(idx_hbm.at[pl.ds(b0, RPW)], idx_v)
        for h in range(RPW // 16):
            rows16 = 16 * h + lax.iota(jnp.int32, 16)
            for s in range(sch):
                v = plsc.load_gather(
                    idx_v, [rows16, jnp.full((16,), s0 + s, jnp.int32)])
                idxT_v[s, 16 * h:16 * h + 16] = v

        def gather(s, slot):
            pltpu.async_copy(table_hbm.at[idxT_v.at[s]], rows_v.at[slot],
                             gsem[slot])

        def wait_gather(slot):
            pltpu.make_async_copy(table_hbm.at[idxT_v.at[0]], rows_v.at[slot],
                                  gsem[slot]).wait()

        def write(s, slot):
            pltpu.async_copy(rows_v.at[slot], g_hbm.at[s, pl.ds(b0, RPW)],
                             wsem[slot])

        def wait_write(slot):
            pltpu.make_async_copy(rows_v.at[slot], g_hbm.at[0, pl.ds(b0, RPW)],
                                  wsem[slot]).wait()

        gather(0, 0)
        gather(1, 1)
        wait_gather(0)
        write(0, 0)

        def step(p, carry):
            s = 2 * p
            wait_write(0)
            gather(s, 0)
            wait_gather(1)
            write(s - 1, 1)
            wait_write(1)
            gather(s + 1, 1)
            wait_gather(0)
            write(s, 0)
            return carry

        lax.fori_loop(1, sch // 2, step, 0)

        wait_gather(1)
        write(sch - 1, 1)
        wait_write(0)
        wait_write(1)

    return _sc_embed


def _proj_body_first(w_ref, b_ref, g_ref, y_ref):
    g = g_ref[0].astype(jnp.bfloat16)
    acc = lax.dot_general(
        w_ref[...], g,
        dimension_numbers=(((1,), (1,)), ((), ())),
        preferred_element_type=jnp.float32,
    )
    y_ref[0] = acc + b_ref[...]


def _proj_body_next(w_ref, b_ref, g_ref, yin_ref, y_ref):
    del yin_ref
    _proj_body_first(w_ref, b_ref, g_ref, y_ref)


def _make_project(s0, sch, first):
    in_specs = [
        pl.BlockSpec((VOCAB, EMBED_DIM), lambda s: (0, 0)),
        pl.BlockSpec((VOCAB, 1), lambda s: (0, 0)),
        pl.BlockSpec((1, BATCH, EMBED_DIM), lambda s: (s, 0, 0)),
    ]
    kwargs = {}
    if not first:
        in_specs.append(pl.BlockSpec(memory_space=pl.ANY))
        kwargs["input_output_aliases"] = {3: 0}
    return pl.pallas_call(
        _proj_body_first if first else _proj_body_next,
        grid=(sch,),
        in_specs=in_specs,
        out_specs=pl.BlockSpec((1, VOCAB, BATCH), lambda s: (s0 + s, 0, 0)),
        out_shape=jax.ShapeDtypeStruct((SEQ, VOCAB, BATCH), jnp.float32),
        **kwargs,
    )


_SC_KERNELS = [_make_sc_embed(s0, sch) for s0, sch in CHUNKS]
_TC_KERNELS = [_make_project(s0, sch, i == 0)
               for i, (s0, sch) in enumerate(CHUNKS)]


@jax.jit
def kernel(x, table, W, b):
    w16 = W.astype(jnp.bfloat16)
    b2d = b.reshape(VOCAB, 1)
    gs = [sck(table, x) for sck in _SC_KERNELS]
    y = _TC_KERNELS[0](w16, b2d, gs[0])
    for i in range(1, len(CHUNKS)):
        y = _TC_KERNELS[i](w16, b2d, gs[i], y)
    return jnp.transpose(y, (2, 0, 1))
